# baseline (speedup 1.0000x reference)
import jax
import jax.numpy as jnp
from jax.experimental import pallas as pl
from jax.experimental.pallas import tpu as pltpu

LAYERS = 6
HEADS = 8
LH = LAYERS * HEADS
L = 2048
T = 512


def _scan_kernel(x_ref, score_ref, hist_ref):
    x = x_ref[0]
    rmax = jnp.max(x, axis=-1, keepdims=True)
    score_ref[0] = jnp.mean(rmax, axis=0, keepdims=True)
    iota_t = jax.lax.broadcasted_iota(jnp.int32, (L, T), 1)
    ridx = jnp.min(jnp.where(x == rmax, iota_t, T), axis=-1, keepdims=True)
    eq = (ridx == iota_t).astype(jnp.int32)
    hist_ref[0, 0] = jnp.sum(eq, axis=0)


def _finalize_kernel(score_ref, hist_ref, dur_ref, focus_ref):
    scores = score_ref[:, :, 0]
    smax = jnp.max(scores, axis=(0, 1), keepdims=True)
    focus_ref[:, :] = smax
    iota_h = jax.lax.broadcasted_iota(jnp.int32, (LH, 1), 0)
    widx = jnp.min(jnp.where(scores == smax, iota_h, LH),
                   axis=(0, 1), keepdims=True)
    mask = (iota_h == widx).astype(jnp.int32)
    dur_ref[0, :] = jnp.sum(hist_ref[:, 0, :] * mask, axis=0)


def kernel(att_ws):
    a = att_ws.reshape(LH, L, T)
    scores, hists = pl.pallas_call(
        _scan_kernel,
        grid=(LH,),
        in_specs=[pl.BlockSpec((1, L, T), lambda i: (i, 0, 0))],
        out_specs=[
            pl.BlockSpec((1, 1, 1), lambda i: (i, 0, 0)),
            pl.BlockSpec((1, 1, T), lambda i: (i, 0, 0)),
        ],
        out_shape=[
            jax.ShapeDtypeStruct((LH, 1, 1), jnp.float32),
            jax.ShapeDtypeStruct((LH, 1, T), jnp.int32),
        ],
    )(a)
    durations, focus = pl.pallas_call(
        _finalize_kernel,
        in_specs=[
            pl.BlockSpec((LH, 1, 1), lambda: (0, 0, 0)),
            pl.BlockSpec((LH, 1, T), lambda: (0, 0, 0)),
        ],
        out_specs=[
            pl.BlockSpec((1, T), lambda: (0, 0)),
            pl.BlockSpec((1, 1), lambda: (0, 0)),
        ],
        out_shape=[
            jax.ShapeDtypeStruct((1, T), jnp.int32),
            jax.ShapeDtypeStruct((1, 1), jnp.float32),
        ],
    )(scores, hists)
    return durations.reshape(T), focus.reshape(())


# speedup vs baseline: 1.2765x; 1.2765x over previous
import jax
import jax.numpy as jnp
from jax.experimental import pallas as pl
from jax.experimental.pallas import tpu as pltpu

LAYERS = 6
HEADS = 8
LH = LAYERS * HEADS
L = 2048
T = 512


def _scan_kernel(x_ref, score_ref):
    rmax = jnp.max(x_ref[0], axis=-1, keepdims=True)
    score_ref[0] = jnp.mean(rmax, axis=0, keepdims=True)


def _finalize_kernel(widx_ref, x_ref, score_ref, dur_ref, focus_ref):
    del widx_ref
    x = x_ref[0]
    rmax = jnp.max(x, axis=-1, keepdims=True)
    iota_t = jax.lax.broadcasted_iota(jnp.int32, (L, T), 1)
    ridx = jnp.min(jnp.where(x == rmax, iota_t, T), axis=-1, keepdims=True)
    eq = (ridx == iota_t).astype(jnp.int32)
    dur_ref[0, :] = jnp.sum(eq, axis=0)
    scores = score_ref[:, :, 0]
    focus_ref[:, :] = jnp.max(scores, axis=(0, 1), keepdims=True)


def kernel(att_ws):
    a = att_ws.reshape(LH, L, T)
    scores = pl.pallas_call(
        _scan_kernel,
        grid=(LH,),
        in_specs=[pl.BlockSpec((1, L, T), lambda i: (i, 0, 0))],
        out_specs=pl.BlockSpec((1, 1, 1), lambda i: (i, 0, 0)),
        out_shape=jax.ShapeDtypeStruct((LH, 1, 1), jnp.float32),
    )(a)
    widx = jnp.argmax(scores.reshape(LH)).astype(jnp.int32).reshape(1)
    durations, focus = pl.pallas_call(
        _finalize_kernel,
        grid_spec=pltpu.PrefetchScalarGridSpec(
            num_scalar_prefetch=1,
            grid=(1,),
            in_specs=[
                pl.BlockSpec((1, L, T), lambda i, w: (w[0], 0, 0)),
                pl.BlockSpec((LH, 1, 1), lambda i, w: (0, 0, 0)),
            ],
            out_specs=[
                pl.BlockSpec((1, T), lambda i, w: (0, 0)),
                pl.BlockSpec((1, 1), lambda i, w: (0, 0)),
            ],
        ),
        out_shape=[
            jax.ShapeDtypeStruct((1, T), jnp.int32),
            jax.ShapeDtypeStruct((1, 1), jnp.float32),
        ],
    )(widx, a, scores)
    return durations.reshape(T), focus.reshape(())


# speedup vs baseline: 1.3612x; 1.0663x over previous
import jax
import jax.numpy as jnp
from jax.experimental import pallas as pl
from jax.experimental.pallas import tpu as pltpu

LAYERS = 6
HEADS = 8
LH = LAYERS * HEADS
L = 2048
T = 512


SCAN_BLOCK = 4


def _scan_kernel(x_ref, score_ref):
    rmax = jnp.max(x_ref[...], axis=-1, keepdims=True)
    score_ref[...] = jnp.mean(rmax, axis=1, keepdims=True)


def _finalize_kernel(widx_ref, x_ref, score_ref, dur_ref, focus_ref):
    del widx_ref
    x = x_ref[0]
    rmax = jnp.max(x, axis=-1, keepdims=True)
    iota_t = jax.lax.broadcasted_iota(jnp.int32, (L, T), 1)
    ridx = jnp.min(jnp.where(x == rmax, iota_t, T), axis=-1, keepdims=True)
    eq = (ridx == iota_t).astype(jnp.int32)
    dur_ref[0, :] = jnp.sum(eq, axis=0)
    scores = score_ref[:, :, 0]
    focus_ref[:, :] = jnp.max(scores, axis=(0, 1), keepdims=True)


def kernel(att_ws):
    a = att_ws.reshape(LH, L, T)
    scores = pl.pallas_call(
        _scan_kernel,
        grid=(LH // SCAN_BLOCK,),
        in_specs=[pl.BlockSpec((SCAN_BLOCK, L, T), lambda i: (i, 0, 0))],
        out_specs=pl.BlockSpec((SCAN_BLOCK, 1, 1), lambda i: (i, 0, 0)),
        out_shape=jax.ShapeDtypeStruct((LH, 1, 1), jnp.float32),
    )(a)
    widx = jnp.argmax(scores.reshape(LH)).astype(jnp.int32).reshape(1)
    durations, focus = pl.pallas_call(
        _finalize_kernel,
        grid_spec=pltpu.PrefetchScalarGridSpec(
            num_scalar_prefetch=1,
            grid=(1,),
            in_specs=[
                pl.BlockSpec((1, L, T), lambda i, w: (w[0], 0, 0)),
                pl.BlockSpec((LH, 1, 1), lambda i, w: (0, 0, 0)),
            ],
            out_specs=[
                pl.BlockSpec((1, T), lambda i, w: (0, 0)),
                pl.BlockSpec((1, 1), lambda i, w: (0, 0)),
            ],
        ),
        out_shape=[
            jax.ShapeDtypeStruct((1, T), jnp.int32),
            jax.ShapeDtypeStruct((1, 1), jnp.float32),
        ],
    )(widx, a, scores)
    return durations.reshape(T), focus.reshape(())
